# Initial kernel scaffold; baseline (speedup 1.0000x reference)
import jax
import jax.numpy as jnp
from jax.experimental import pallas as pl


def kernel(seq_len, rel_k_table, rel_v_table):
    raise NotImplementedError("write your pallas kernel here")


# speedup vs baseline: 8.5517x; 8.5517x over previous
import jax
import jax.numpy as jnp
from jax import lax
from jax.experimental import pallas as pl
from jax.experimental.pallas import tpu as pltpu
from jax.experimental.pallas import tpu_sc as plsc

S = 2048
D = 32
MAX_REL = 128
NC, NS = 2, 16
NW = NC * NS
RPW = S // NW
WIN = S + RPW


def _rpe_body(tk, tv, out_k, out_v, tbl, win, sem):
    wid = lax.axis_index("s") * NC + lax.axis_index("c")
    g0 = (S - RPW) - RPW * wid

    def one_table(table, out):
        pltpu.async_copy(table, tbl, sem).wait()

        def fill(p, _):
            t = jnp.clip(g0 + p - (S - 1), -MAX_REL, MAX_REL) + MAX_REL
            win[p, pl.ds(0, 16)] = tbl[t, pl.ds(0, 16)]
            win[p, pl.ds(16, 16)] = tbl[t, pl.ds(16, 16)]
            return _
        lax.fori_loop(0, WIN, fill, None)

        def write(r, _):
            i = RPW * wid + r
            pltpu.async_copy(
                win.at[pl.ds(RPW - 1 - r, S)], out.at[i], sem
            ).wait()
            return _
        lax.fori_loop(0, RPW, write, None)

    one_table(tk, out_k)
    one_table(tv, out_v)


def kernel(seq_len, rel_k_table, rel_v_table):
    del seq_len
    f = pl.kernel(
        _rpe_body,
        out_type=(
            jax.ShapeDtypeStruct((S, S, D), jnp.float32),
            jax.ShapeDtypeStruct((S, S, D), jnp.float32),
        ),
        mesh=plsc.VectorSubcoreMesh(core_axis_name="c", subcore_axis_name="s"),
        compiler_params=pltpu.CompilerParams(use_tc_tiling_on_sc=False),
        scratch_types=[
            pltpu.VMEM((2 * MAX_REL + 1, D), jnp.float32),
            pltpu.VMEM((WIN, D), jnp.float32),
            pltpu.SemaphoreType.DMA,
        ],
    )
    return tuple(f(rel_k_table, rel_v_table))


# speedup vs baseline: 17.4728x; 2.0432x over previous
import jax
import jax.numpy as jnp
from jax import lax
from jax.experimental import pallas as pl
from jax.experimental.pallas import tpu as pltpu
from jax.experimental.pallas import tpu_sc as plsc

S = 2048
D = 32
MAX_REL = 128
NC, NS = 2, 16
NW = NC * NS
RPW = S // NW
PR = S * D // 128
WINQ = 576


def _rpe_body(tk, tv, out_k, out_v, tbl, win, sem):
    wid = lax.axis_index("s") * NC + lax.axis_index("c")
    a = wid % 4
    wsub = wid // 4
    b0 = (S - 253) - a - 256 * wsub

    def one_table(table, out):
        pltpu.async_copy(table, tbl, sem).wait()

        def fill(q, _):
            g = b0 + 4 * q
            for c in range(4):
                t = jnp.clip(g + c - (S - 1), -MAX_REL, MAX_REL) + MAX_REL
                win[q, pl.ds(32 * c, 16)] = tbl[t, pl.ds(0, 16)]
                win[q, pl.ds(32 * c + 16, 16)] = tbl[t, pl.ds(16, 16)]
            return _
        lax.fori_loop(0, WINQ, fill, None)

        def write(m, _):
            i = a + 256 * wsub + 4 * m
            pltpu.async_copy(
                win.at[pl.ds(RPW - 1 - m, PR)], out.at[i], sem
            ).wait()
            return _
        lax.fori_loop(0, RPW, write, None)

    one_table(tk, out_k)
    one_table(tv, out_v)


def kernel(seq_len, rel_k_table, rel_v_table):
    del seq_len
    f = pl.kernel(
        _rpe_body,
        out_type=(
            jax.ShapeDtypeStruct((S, PR, 128), jnp.float32),
            jax.ShapeDtypeStruct((S, PR, 128), jnp.float32),
        ),
        mesh=plsc.VectorSubcoreMesh(core_axis_name="c", subcore_axis_name="s"),
        compiler_params=pltpu.CompilerParams(use_tc_tiling_on_sc=False),
        scratch_types=[
            pltpu.VMEM((2 * MAX_REL + 1, D), jnp.float32),
            pltpu.VMEM((WINQ, 128), jnp.float32),
            pltpu.SemaphoreType.DMA,
        ],
    )
    ok, ov = f(rel_k_table, rel_v_table)
    return (ok.reshape(S, S, D), ov.reshape(S, S, D))


# speedup vs baseline: 75.3071x; 4.3100x over previous
import jax
import jax.numpy as jnp
from jax import lax
from jax.experimental import pallas as pl
from jax.experimental.pallas import tpu as pltpu
from jax.experimental.pallas import tpu_sc as plsc

S = 2048
D = 32
MAX_REL = 128
NC, NS = 2, 16
WQ = 31
BQ = 14


def _rpe_body(tk, tv, out_k, out_v, tbl, win, sem):
    wid = lax.axis_index("s") * NC + lax.axis_index("c")
    h = wid % 2
    G = wid // 2
    lane = lax.iota(jnp.int32, 16)

    def one_table(table, out):
        pltpu.async_copy(table, tbl, sem).wait()

        for dtl in range(2):
            for ds_ in range(8):
                dvec = jnp.full((16,), 16 * h + 8 * dtl + ds_, jnp.int32)
                v0 = plsc.load_gather(
                    tbl, [jnp.zeros((16,), jnp.int32), dvec]
                )
                v1 = plsc.load_gather(
                    tbl, [jnp.full((16,), 2 * MAX_REL, jnp.int32), dvec]
                )

                def cfill(n, _, dtl=dtl, ds_=ds_, v0=v0, v1=v1):
                    q = n // 8
                    lc = n % 8
                    win[dtl, q, ds_, pl.ds(16 * lc, 16)] = v0
                    win[dtl, q + 17, ds_, pl.ds(16 * lc, 16)] = v1
                    return _
                lax.fori_loop(0, BQ * 8, cfill, None)

        def do_class(j, _):
            a = G + 16 * j

            def bfill(n, _):
                dtl = n // 192
                r = n % 192
                q = BQ + r // 64
                ds_ = (r % 64) // 8
                lc = r % 8
                c0 = (127 - a) + 128 * q + 16 * lc
                tidx = (
                    jnp.clip(c0 + lane - (S - 1), -MAX_REL, MAX_REL) + MAX_REL
                )
                dvec = jnp.full((16,), 16 * h + 8 * dtl + ds_, jnp.int32)
                win[dtl, q, ds_, pl.ds(16 * lc, 16)] = plsc.load_gather(
                    tbl, [tidx, dvec]
                )
                return _
            lax.fori_loop(0, 2 * 3 * 64, bfill, None)

            def wrow(t, _):
                i = a + 128 * t
                pltpu.async_copy(
                    win.at[:, pl.ds(15 - t, 16)],
                    out.at[i, pl.ds(2 * h, 2)],
                    sem,
                ).wait()
                return _
            lax.fori_loop(0, 16, wrow, None)
            return _
        lax.fori_loop(0, 8, do_class, None)

    one_table(tk, out_k)
    one_table(tv, out_v)


def kernel(seq_len, rel_k_table, rel_v_table):
    del seq_len
    f = pl.kernel(
        _rpe_body,
        out_type=(
            jax.ShapeDtypeStruct((S, 4, 16, 8, 128), jnp.float32),
            jax.ShapeDtypeStruct((S, 4, 16, 8, 128), jnp.float32),
        ),
        mesh=plsc.VectorSubcoreMesh(core_axis_name="c", subcore_axis_name="s"),
        compiler_params=pltpu.CompilerParams(
            use_tc_tiling_on_sc=False, needs_layout_passes=False
        ),
        scratch_types=[
            pltpu.VMEM((2 * MAX_REL + 1, D), jnp.float32),
            pltpu.VMEM((2, WQ, 8, 128), jnp.float32),
            pltpu.SemaphoreType.DMA,
        ],
    )
    ok, ov = f(rel_k_table, rel_v_table)
    ok = ok.transpose(0, 2, 4, 1, 3).reshape(S, S, D)
    ov = ov.transpose(0, 2, 4, 1, 3).reshape(S, S, D)
    return (ok, ov)


# speedup vs baseline: 83.6501x; 1.1108x over previous
import jax
import jax.numpy as jnp
from jax import lax
from jax.experimental import pallas as pl
from jax.experimental.pallas import tpu as pltpu
from jax.experimental.pallas import tpu_sc as plsc

S = 2048
D = 32
MAX_REL = 128
NC, NS = 2, 16
WQ = 31
BQ = 14


def _rpe_body(tk, tv, out_k, out_v, tbl, win, sb, sem):
    wid = lax.axis_index("s") * NC + lax.axis_index("c")
    h = wid % 2
    G = wid // 2
    lane = lax.iota(jnp.int32, 16)

    def one_table(table, out):
        pltpu.async_copy(table, tbl, sem).wait()

        for dtl in range(2):
            for ds_ in range(8):
                dvec = jnp.full((16,), 16 * h + 8 * dtl + ds_, jnp.int32)
                v0 = plsc.load_gather(
                    tbl, [jnp.zeros((16,), jnp.int32), dvec]
                )
                v1 = plsc.load_gather(
                    tbl, [jnp.full((16,), 2 * MAX_REL, jnp.int32), dvec]
                )

                def cfill(n, _, dtl=dtl, ds_=ds_, v0=v0, v1=v1):
                    q = n // 8
                    lc = n % 8
                    win[dtl, q, ds_, pl.ds(16 * lc, 16)] = v0
                    win[dtl, q + 17, ds_, pl.ds(16 * lc, 16)] = v1
                    return _
                lax.fori_loop(0, BQ * 8, cfill, None)

        def bfill(dst, qoff, a):
            def one(n, _):
                dtl = n // 192
                r = n % 192
                qb = r // 64
                ds_ = (r % 64) // 8
                lc = r % 8
                c0 = (127 - a) + 128 * (BQ + qb) + 16 * lc
                tidx = (
                    jnp.clip(c0 + lane - (S - 1), -MAX_REL, MAX_REL) + MAX_REL
                )
                dvec = jnp.full((16,), 16 * h + 8 * dtl + ds_, jnp.int32)
                dst[dtl, qoff + qb, ds_, pl.ds(16 * lc, 16)] = plsc.load_gather(
                    tbl, [tidx, dvec]
                )
                return _
            lax.fori_loop(0, 2 * 3 * 64, one, None)

        bfill(win, BQ, G)

        def do_class(j, _):
            a = G + 16 * j

            def fire(t, _):
                i = a + 128 * t
                pltpu.make_async_copy(
                    win.at[:, pl.ds(15 - t, 16)],
                    out.at[i, pl.ds(2 * h, 2)],
                    sem,
                ).start()
                return _
            lax.fori_loop(0, 16, fire, None)

            @pl.when(j < 7)
            def _stage():
                bfill(sb, 0, a + 16)

            def drain(t, _):
                pltpu.make_async_copy(
                    win.at[:, pl.ds(0, 16)],
                    out.at[a, pl.ds(2 * h, 2)],
                    sem,
                ).wait()
                return _
            lax.fori_loop(0, 16, drain, None)

            @pl.when(j < 7)
            def _commit():
                def cp(n, _):
                    dtl = n // 24
                    r = n % 24
                    qb = r // 8
                    lc = r % 8
                    def cps(ds_, _, dtl=dtl, qb=qb, lc=lc):
                        win[dtl, BQ + qb, ds_, pl.ds(16 * lc, 16)] = sb[
                            dtl, qb, ds_, pl.ds(16 * lc, 16)
                        ]
                        return _
                    lax.fori_loop(0, 8, cps, None)
                    return _
                lax.fori_loop(0, 2 * 3 * 8, cp, None)
            return _
        lax.fori_loop(0, 8, do_class, None)

    one_table(tk, out_k)
    one_table(tv, out_v)


def kernel(seq_len, rel_k_table, rel_v_table):
    del seq_len
    f = pl.kernel(
        _rpe_body,
        out_type=(
            jax.ShapeDtypeStruct((S, 4, 16, 8, 128), jnp.float32),
            jax.ShapeDtypeStruct((S, 4, 16, 8, 128), jnp.float32),
        ),
        mesh=plsc.VectorSubcoreMesh(core_axis_name="c", subcore_axis_name="s"),
        compiler_params=pltpu.CompilerParams(
            use_tc_tiling_on_sc=False, needs_layout_passes=False
        ),
        scratch_types=[
            pltpu.VMEM((2 * MAX_REL + 1, D), jnp.float32),
            pltpu.VMEM((2, WQ, 8, 128), jnp.float32),
            pltpu.VMEM((2, 3, 8, 128), jnp.float32),
            pltpu.SemaphoreType.DMA,
        ],
    )
    ok, ov = f(rel_k_table, rel_v_table)
    ok = ok.transpose(0, 2, 4, 1, 3).reshape(S, S, D)
    ov = ov.transpose(0, 2, 4, 1, 3).reshape(S, S, D)
    return (ok, ov)
